# baseline (speedup 1.0000x reference)
import functools

import jax
import jax.numpy as jnp
from jax import lax
from jax.experimental import pallas as pl
from jax.experimental.pallas import tpu as pltpu
from jax.experimental.pallas import tpu_sc as plsc

N = 10000
E = 320000
D = 128
H = 8
C = 16

NC = 2
NS = 16
HC = H // NC
HW = HC * C

CH = 80
EP = E // NS
NCHUNK = EP // CH
N_PAD = 10240
ROWS_PER_SUB = N_PAD // NS

BN = 1000


def _edge_kernel(q_hbm, kv_hbm, src_hbm, dst_hbm, out_hbm,
                 src_v0, dst_v0, src_v1, dst_v1, q_v0, kv_v0, q_v1, kv_v1,
                 msg_v, acc_sh, sq0, skv0, sq1, skv1):
    cid = lax.axis_index("c")
    sid = lax.axis_index("s")

    lane = lax.iota(jnp.int32, 16)
    zeros16 = jnp.zeros((16,), jnp.float32)

    def zmsg(i, carry):
        for j in range(D // 16):
            msg_v[i, pl.ds(j * 16, 16)] = zeros16
        return carry
    lax.fori_loop(0, CH, zmsg, 0)

    for j in range(ROWS_PER_SUB // CH):
        pltpu.sync_copy(msg_v, acc_sh.at[pl.ds(sid * ROWS_PER_SUB + j * CH, CH)])
    plsc.subcore_barrier()

    base_e = sid * EP
    qc = q_hbm.at[cid]
    kvc = kv_hbm.at[cid]

    def fetch_idx(c, sv, dv):
        off = base_e + c * CH
        pltpu.sync_copy(src_hbm.at[pl.ds(off, CH)], sv)
        pltpu.sync_copy(dst_hbm.at[pl.ds(off, CH)], dv)

    def issue(sv, dv, qv, kvv, s1, s2):
        pltpu.async_copy(qc.at[dv], qv, s1)
        pltpu.async_copy(kvc.at[sv], kvv, s2)

    def wait(dv, sv, qv, kvv, s1, s2):
        pltpu.make_async_copy(qc.at[dv], qv, s1).wait()
        pltpu.make_async_copy(kvc.at[sv], kvv, s2).wait()

    def compute(qv, kvv, dv):
        @plsc.parallel_loop(0, CH, step=1, unroll=16)
        def edge_body(e):
            exv = zeros16
            for h in range(HC):
                qh = qv[e, pl.ds(h * 16, 16)]
                kh = kvv[e, pl.ds(h * 16, 16)]
                s = jnp.sum(qh * kh)
                ex = jnp.exp(jnp.broadcast_to(s, (16,)))
                vh = kvv[e, pl.ds(HW + h * 16, 16)]
                msg_v[e, pl.ds(h * 16, 16)] = ex * vh
                exv = jnp.where(lane == h, ex, exv)
            msg_v[e, pl.ds(HW, 16)] = exv
        pltpu.sync_copy(msg_v, acc_sh.at[dv], add=True)

    fetch_idx(0, src_v0, dst_v0)
    issue(src_v0, dst_v0, q_v0, kv_v0, sq0, skv0)

    NPAIR = NCHUNK // 2

    def pair_body(i, carry):
        c0 = 2 * i
        fetch_idx(c0 + 1, src_v1, dst_v1)
        issue(src_v1, dst_v1, q_v1, kv_v1, sq1, skv1)
        wait(dst_v0, src_v0, q_v0, kv_v0, sq0, skv0)
        compute(q_v0, kv_v0, dst_v0)

        @pl.when(i < NPAIR - 1)
        def _():
            fetch_idx(c0 + 2, src_v0, dst_v0)
            issue(src_v0, dst_v0, q_v0, kv_v0, sq0, skv0)

        wait(dst_v1, src_v1, q_v1, kv_v1, sq1, skv1)
        compute(q_v1, kv_v1, dst_v1)
        return carry
    lax.fori_loop(0, NPAIR, pair_body, 0)

    plsc.subcore_barrier()

    for j in range(ROWS_PER_SUB // CH):
        r = sid * ROWS_PER_SUB + j * CH
        pltpu.sync_copy(acc_sh.at[pl.ds(r, CH)], msg_v)
        pltpu.sync_copy(msg_v, out_hbm.at[cid, pl.ds(r, CH)])


_edge_call = functools.partial(
    pl.kernel,
    out_type=jax.ShapeDtypeStruct((NC, N_PAD, D), jnp.float32),
    mesh=plsc.VectorSubcoreMesh(core_axis_name="c", subcore_axis_name="s"),
    compiler_params=pltpu.CompilerParams(needs_layout_passes=False, use_tc_tiling_on_sc=False),
    scratch_types=[
        pltpu.VMEM((CH,), jnp.int32),
        pltpu.VMEM((CH,), jnp.int32),
        pltpu.VMEM((CH,), jnp.int32),
        pltpu.VMEM((CH,), jnp.int32),
        pltpu.VMEM((CH, HW), jnp.float32),
        pltpu.VMEM((CH, 2 * HW), jnp.float32),
        pltpu.VMEM((CH, HW), jnp.float32),
        pltpu.VMEM((CH, 2 * HW), jnp.float32),
        pltpu.VMEM((CH, D), jnp.float32),
        pltpu.VMEM_SHARED((N_PAD, D), jnp.float32),
        pltpu.SemaphoreType.DMA,
        pltpu.SemaphoreType.DMA,
        pltpu.SemaphoreType.DMA,
        pltpu.SemaphoreType.DMA,
    ],
)(_edge_kernel)


def _split_heads(q, kv):
    qa = q[:, :HW]
    qb = q[:, HW:]
    kva = jnp.concatenate([kv[:, :HW], kv[:, D:D + HW]], axis=1)
    kvb = jnp.concatenate([kv[:, HW:D], kv[:, D + HW:]], axis=1)
    return qa, qb, kva, kvb


def _proj_body(x_ref, wq_ref, wkv_ref, ws_ref, bq_ref, bkv_ref, bs_ref,
               qa_ref, qb_ref, kva_ref, kvb_ref, xs_ref):
    x = x_ref[...]
    q = jnp.dot(x, wq_ref[...], preferred_element_type=jnp.float32) + bq_ref[...]
    kv = jnp.dot(x, wkv_ref[...], preferred_element_type=jnp.float32) + bkv_ref[...]
    qa, qb, kva, kvb = _split_heads(q, kv)
    qa_ref[...] = qa
    qb_ref[...] = qb
    kva_ref[...] = kva
    kvb_ref[...] = kvb
    xs_ref[...] = jnp.dot(x, ws_ref[...], preferred_element_type=jnp.float32) + bs_ref[...]


_PROJ_OUT_SPECS = [
    pl.BlockSpec((BN, HW), lambda i: (i, 0)),
    pl.BlockSpec((BN, HW), lambda i: (i, 0)),
    pl.BlockSpec((BN, 2 * HW), lambda i: (i, 0)),
    pl.BlockSpec((BN, 2 * HW), lambda i: (i, 0)),
    pl.BlockSpec((BN, D), lambda i: (i, 0)),
]
_PROJ_OUT_SHAPE = [
    jax.ShapeDtypeStruct((N, HW), jnp.float32),
    jax.ShapeDtypeStruct((N, HW), jnp.float32),
    jax.ShapeDtypeStruct((N, 2 * HW), jnp.float32),
    jax.ShapeDtypeStruct((N, 2 * HW), jnp.float32),
    jax.ShapeDtypeStruct((N, D), jnp.float32),
]
_W_SPECS = [
    pl.BlockSpec((D, D), lambda i: (0, 0)),
    pl.BlockSpec((D, 2 * D), lambda i: (0, 0)),
    pl.BlockSpec((D, D), lambda i: (0, 0)),
    pl.BlockSpec((1, D), lambda i: (0, 0)),
    pl.BlockSpec((1, 2 * D), lambda i: (0, 0)),
    pl.BlockSpec((1, D), lambda i: (0, 0)),
]


def _proj_tc(x, wq_t, wkv_t, ws_t, bq, bkv, bs):
    return pl.pallas_call(
        _proj_body,
        grid=(N // BN,),
        in_specs=[pl.BlockSpec((BN, D), lambda i: (i, 0))] + _W_SPECS,
        out_specs=_PROJ_OUT_SPECS,
        out_shape=_PROJ_OUT_SHAPE,
    )(x, wq_t, wkv_t, ws_t, bq, bkv, bs)


def _attention_out(acc_ref, xs_ref, expand_ref, a_ref):
    acc0 = acc_ref[0]
    acc1 = acc_ref[1]
    msg = jnp.concatenate([acc0[:, :HW], acc1[:, :HW]], axis=1)
    den = jnp.concatenate([acc0[:, HW:HW + HC], acc1[:, HW:HW + HC]], axis=1)
    den_b = jnp.dot(den, expand_ref[...], preferred_element_type=jnp.float32)
    o = msg / (den_b + 1e-16) + xs_ref[...]
    return jnp.where(o >= 0.0, o, o * a_ref[...])


def _fin1_body(acc_ref, xs_ref, expand_ref, a_ref,
               wq_ref, wkv_ref, ws_ref, bq_ref, bkv_ref, bs_ref,
               qa_ref, qb_ref, kva_ref, kvb_ref, xs2_ref):
    out1 = _attention_out(acc_ref, xs_ref, expand_ref, a_ref)
    q = jnp.dot(out1, wq_ref[...], preferred_element_type=jnp.float32) + bq_ref[...]
    kv = jnp.dot(out1, wkv_ref[...], preferred_element_type=jnp.float32) + bkv_ref[...]
    qa, qb, kva, kvb = _split_heads(q, kv)
    qa_ref[...] = qa
    qb_ref[...] = qb
    kva_ref[...] = kva
    kvb_ref[...] = kvb
    xs2_ref[...] = jnp.dot(out1, ws_ref[...], preferred_element_type=jnp.float32) + bs_ref[...]


def _fin1_tc(acc, xs, expand, a2d, wq_t, wkv_t, ws_t, bq, bkv, bs):
    return pl.pallas_call(
        _fin1_body,
        grid=(N // BN,),
        in_specs=[
            pl.BlockSpec((NC, BN, D), lambda i: (0, i, 0)),
            pl.BlockSpec((BN, D), lambda i: (i, 0)),
            pl.BlockSpec((H, D), lambda i: (0, 0)),
            pl.BlockSpec((1, D), lambda i: (0, 0)),
        ] + _W_SPECS,
        out_specs=_PROJ_OUT_SPECS,
        out_shape=_PROJ_OUT_SHAPE,
    )(acc, xs, expand, a2d, wq_t, wkv_t, ws_t, bq, bkv, bs)


def _fin2_body(acc_ref, xs_ref, expand_ref, a_ref, x_ref, out_ref):
    acc0 = acc_ref[0]
    acc1 = acc_ref[1]
    msg = jnp.concatenate([acc0[:, :HW], acc1[:, :HW]], axis=1)
    den = jnp.concatenate([acc0[:, HW:HW + HC], acc1[:, HW:HW + HC]], axis=1)
    den_b = jnp.dot(den, expand_ref[...], preferred_element_type=jnp.float32)
    o = msg / (den_b + 1e-16) + xs_ref[...] + x_ref[...]
    out_ref[...] = jnp.where(o >= 0.0, o, o * a_ref[...])


def _fin2_tc(acc, xs, expand, a2d, x):
    return pl.pallas_call(
        _fin2_body,
        grid=(N // BN,),
        in_specs=[
            pl.BlockSpec((NC, BN, D), lambda i: (0, i, 0)),
            pl.BlockSpec((BN, D), lambda i: (i, 0)),
            pl.BlockSpec((H, D), lambda i: (0, 0)),
            pl.BlockSpec((1, D), lambda i: (0, 0)),
            pl.BlockSpec((BN, D), lambda i: (i, 0)),
        ],
        out_specs=pl.BlockSpec((BN, D), lambda i: (i, 0)),
        out_shape=jax.ShapeDtypeStruct((N, D), jnp.float32),
    )(acc, xs, expand, a2d, x)


def kernel(x, edge_index, Wq1, bq1, Wk1, bk1, Wv1, bv1, Ws1, bs1,
           Wq2, bq2, Wk2, bk2, Wv2, bv2, Ws2, bs2, a):
    inv_sqrt_c = 1.0 / (C ** 0.5)
    src = edge_index[0]
    dst = edge_index[1]

    wq1_t = (Wq1.T * inv_sqrt_c).astype(jnp.float32)
    wkv1_t = jnp.concatenate([Wk1.T, Wv1.T], axis=1)
    ws1_t = Ws1.T
    bq1s = (bq1 * inv_sqrt_c).reshape(1, D)
    bkv1 = jnp.concatenate([bk1, bv1]).reshape(1, 2 * D)
    bs1r = bs1.reshape(1, D)

    wq2_t = (Wq2.T * inv_sqrt_c).astype(jnp.float32)
    wkv2_t = jnp.concatenate([Wk2.T, Wv2.T], axis=1)
    ws2_t = Ws2.T
    bq2s = (bq2 * inv_sqrt_c).reshape(1, D)
    bkv2 = jnp.concatenate([bk2, bv2]).reshape(1, 2 * D)
    bs2r = bs2.reshape(1, D)

    expand = jnp.kron(jnp.eye(H, dtype=jnp.float32),
                      jnp.ones((1, C), dtype=jnp.float32))
    a2d = jnp.broadcast_to(a.astype(jnp.float32), (1, D))

    qa1, qb1, kva1, kvb1, xs1 = _proj_tc(x, wq1_t, wkv1_t, ws1_t, bq1s, bkv1, bs1r)
    q1 = jnp.stack([qa1, qb1])
    kv1 = jnp.stack([kva1, kvb1])
    acc1 = _edge_call(q1, kv1, src, dst)
    qa2, qb2, kva2, kvb2, xs2 = _fin1_tc(acc1, xs1, expand, a2d,
                                         wq2_t, wkv2_t, ws2_t, bq2s, bkv2, bs2r)
    q2 = jnp.stack([qa2, qb2])
    kv2 = jnp.stack([kva2, kvb2])
    acc2 = _edge_call(q2, kv2, src, dst)
    return _fin2_tc(acc2, xs2, expand, a2d, x)


# speedup vs baseline: 2.8368x; 2.8368x over previous
import functools

import jax
import jax.numpy as jnp
from jax import lax
from jax.experimental import pallas as pl
from jax.experimental.pallas import tpu as pltpu
from jax.experimental.pallas import tpu_sc as plsc

N = 10000
E = 320000
D = 128
H = 8
C = 16

NC = 2
NS = 16
HC = H // NC
HW = HC * C

CH = 80
EP = E // NS
NCHUNK = EP // CH
N_PAD = 10240
ROWS_PER_SUB = N_PAD // NS

BN = 1000


def _edge_kernel(q_hbm, kv_hbm, src_hbm, dst_hbm, out_hbm,
                 src_v0, dst_v0, src_v1, dst_v1, q_v0, kv_v0, q_v1, kv_v1,
                 msg_v, acc_sh, sq0, skv0, sq1, skv1):
    cid = lax.axis_index("c")
    sid = lax.axis_index("s")

    lane = lax.iota(jnp.int32, 16)
    zeros16 = jnp.zeros((16,), jnp.float32)

    def zmsg(i, carry):
        for j in range(D // 16):
            msg_v[i, pl.ds(j * 16, 16)] = zeros16
        return carry
    lax.fori_loop(0, CH, zmsg, 0)

    for j in range(ROWS_PER_SUB // CH):
        pltpu.sync_copy(msg_v, acc_sh.at[pl.ds(sid * ROWS_PER_SUB + j * CH, CH)])
    plsc.subcore_barrier()

    base_e = sid * EP
    qc = q_hbm.at[cid]
    kvc = kv_hbm.at[cid]

    def fetch_idx(c, sv, dv):
        off = base_e + c * CH
        pltpu.sync_copy(src_hbm.at[pl.ds(off, CH)], sv)
        pltpu.sync_copy(dst_hbm.at[pl.ds(off, CH)], dv)

    def issue(sv, dv, qv, kvv, s1, s2):
        pltpu.async_copy(qc.at[dv], qv, s1)
        pltpu.async_copy(kvc.at[sv], kvv, s2)

    def wait(dv, sv, qv, kvv, s1, s2):
        pltpu.make_async_copy(qc.at[dv], qv, s1).wait()
        pltpu.make_async_copy(kvc.at[sv], kvv, s2).wait()

    def compute(qv, kvv, dv):
        @plsc.parallel_loop(0, CH, step=1, unroll=4)
        def edge_body(e):
            exv = zeros16
            for h in range(HC):
                qh = qv[e, pl.ds(h * 16, 16)]
                kh = kvv[e, pl.ds(h * 16, 16)]
                s = jnp.sum(qh * kh)
                ex = jnp.exp(jnp.broadcast_to(s, (16,)))
                vh = kvv[e, pl.ds(HW + h * 16, 16)]
                msg_v[e, pl.ds(h * 16, 16)] = ex * vh
                exv = jnp.where(lane == h, ex, exv)
            msg_v[e, pl.ds(HW, 16)] = exv
        pltpu.sync_copy(msg_v, acc_sh.at[dv], add=True)

    fetch_idx(0, src_v0, dst_v0)
    issue(src_v0, dst_v0, q_v0, kv_v0, sq0, skv0)

    NPAIR = NCHUNK // 2

    def pair_body(i, carry):
        c0 = 2 * i
        fetch_idx(c0 + 1, src_v1, dst_v1)
        issue(src_v1, dst_v1, q_v1, kv_v1, sq1, skv1)
        wait(dst_v0, src_v0, q_v0, kv_v0, sq0, skv0)
        compute(q_v0, kv_v0, dst_v0)

        @pl.when(i < NPAIR - 1)
        def _():
            fetch_idx(c0 + 2, src_v0, dst_v0)
            issue(src_v0, dst_v0, q_v0, kv_v0, sq0, skv0)

        wait(dst_v1, src_v1, q_v1, kv_v1, sq1, skv1)
        compute(q_v1, kv_v1, dst_v1)
        return carry
    lax.fori_loop(0, NPAIR, pair_body, 0)

    plsc.subcore_barrier()

    for j in range(ROWS_PER_SUB // CH):
        r = sid * ROWS_PER_SUB + j * CH
        pltpu.sync_copy(acc_sh.at[pl.ds(r, CH)], msg_v)
        pltpu.sync_copy(msg_v, out_hbm.at[cid, pl.ds(r, CH)])


_edge_call = functools.partial(
    pl.kernel,
    out_type=jax.ShapeDtypeStruct((NC, N_PAD, D), jnp.float32),
    mesh=plsc.VectorSubcoreMesh(core_axis_name="c", subcore_axis_name="s"),
    compiler_params=pltpu.CompilerParams(needs_layout_passes=False, use_tc_tiling_on_sc=False),
    scratch_types=[
        pltpu.VMEM((CH,), jnp.int32),
        pltpu.VMEM((CH,), jnp.int32),
        pltpu.VMEM((CH,), jnp.int32),
        pltpu.VMEM((CH,), jnp.int32),
        pltpu.VMEM((CH, HW), jnp.float32),
        pltpu.VMEM((CH, 2 * HW), jnp.float32),
        pltpu.VMEM((CH, HW), jnp.float32),
        pltpu.VMEM((CH, 2 * HW), jnp.float32),
        pltpu.VMEM((CH, D), jnp.float32),
        pltpu.VMEM_SHARED((N_PAD, D), jnp.float32),
        pltpu.SemaphoreType.DMA,
        pltpu.SemaphoreType.DMA,
        pltpu.SemaphoreType.DMA,
        pltpu.SemaphoreType.DMA,
    ],
)(_edge_kernel)


def _split_heads(q, kv):
    qa = q[:, :HW]
    qb = q[:, HW:]
    kva = jnp.concatenate([kv[:, :HW], kv[:, D:D + HW]], axis=1)
    kvb = jnp.concatenate([kv[:, HW:D], kv[:, D + HW:]], axis=1)
    return qa, qb, kva, kvb


def _proj_body(x_ref, wq_ref, wkv_ref, ws_ref, bq_ref, bkv_ref, bs_ref,
               qa_ref, qb_ref, kva_ref, kvb_ref, xs_ref):
    x = x_ref[...]
    q = jnp.dot(x, wq_ref[...], preferred_element_type=jnp.float32) + bq_ref[...]
    kv = jnp.dot(x, wkv_ref[...], preferred_element_type=jnp.float32) + bkv_ref[...]
    qa, qb, kva, kvb = _split_heads(q, kv)
    qa_ref[...] = qa
    qb_ref[...] = qb
    kva_ref[...] = kva
    kvb_ref[...] = kvb
    xs_ref[...] = jnp.dot(x, ws_ref[...], preferred_element_type=jnp.float32) + bs_ref[...]


_PROJ_OUT_SPECS = [
    pl.BlockSpec((BN, HW), lambda i: (i, 0)),
    pl.BlockSpec((BN, HW), lambda i: (i, 0)),
    pl.BlockSpec((BN, 2 * HW), lambda i: (i, 0)),
    pl.BlockSpec((BN, 2 * HW), lambda i: (i, 0)),
    pl.BlockSpec((BN, D), lambda i: (i, 0)),
]
_PROJ_OUT_SHAPE = [
    jax.ShapeDtypeStruct((N, HW), jnp.float32),
    jax.ShapeDtypeStruct((N, HW), jnp.float32),
    jax.ShapeDtypeStruct((N, 2 * HW), jnp.float32),
    jax.ShapeDtypeStruct((N, 2 * HW), jnp.float32),
    jax.ShapeDtypeStruct((N, D), jnp.float32),
]
_W_SPECS = [
    pl.BlockSpec((D, D), lambda i: (0, 0)),
    pl.BlockSpec((D, 2 * D), lambda i: (0, 0)),
    pl.BlockSpec((D, D), lambda i: (0, 0)),
    pl.BlockSpec((1, D), lambda i: (0, 0)),
    pl.BlockSpec((1, 2 * D), lambda i: (0, 0)),
    pl.BlockSpec((1, D), lambda i: (0, 0)),
]


def _proj_tc(x, wq_t, wkv_t, ws_t, bq, bkv, bs):
    return pl.pallas_call(
        _proj_body,
        grid=(N // BN,),
        in_specs=[pl.BlockSpec((BN, D), lambda i: (i, 0))] + _W_SPECS,
        out_specs=_PROJ_OUT_SPECS,
        out_shape=_PROJ_OUT_SHAPE,
    )(x, wq_t, wkv_t, ws_t, bq, bkv, bs)


def _attention_out(acc_ref, xs_ref, expand_ref, a_ref):
    acc0 = acc_ref[0]
    acc1 = acc_ref[1]
    msg = jnp.concatenate([acc0[:, :HW], acc1[:, :HW]], axis=1)
    den = jnp.concatenate([acc0[:, HW:HW + HC], acc1[:, HW:HW + HC]], axis=1)
    den_b = jnp.dot(den, expand_ref[...], preferred_element_type=jnp.float32)
    o = msg / (den_b + 1e-16) + xs_ref[...]
    return jnp.where(o >= 0.0, o, o * a_ref[...])


def _fin1_body(acc_ref, xs_ref, expand_ref, a_ref,
               wq_ref, wkv_ref, ws_ref, bq_ref, bkv_ref, bs_ref,
               qa_ref, qb_ref, kva_ref, kvb_ref, xs2_ref):
    out1 = _attention_out(acc_ref, xs_ref, expand_ref, a_ref)
    q = jnp.dot(out1, wq_ref[...], preferred_element_type=jnp.float32) + bq_ref[...]
    kv = jnp.dot(out1, wkv_ref[...], preferred_element_type=jnp.float32) + bkv_ref[...]
    qa, qb, kva, kvb = _split_heads(q, kv)
    qa_ref[...] = qa
    qb_ref[...] = qb
    kva_ref[...] = kva
    kvb_ref[...] = kvb
    xs2_ref[...] = jnp.dot(out1, ws_ref[...], preferred_element_type=jnp.float32) + bs_ref[...]


def _fin1_tc(acc, xs, expand, a2d, wq_t, wkv_t, ws_t, bq, bkv, bs):
    return pl.pallas_call(
        _fin1_body,
        grid=(N // BN,),
        in_specs=[
            pl.BlockSpec((NC, BN, D), lambda i: (0, i, 0)),
            pl.BlockSpec((BN, D), lambda i: (i, 0)),
            pl.BlockSpec((H, D), lambda i: (0, 0)),
            pl.BlockSpec((1, D), lambda i: (0, 0)),
        ] + _W_SPECS,
        out_specs=_PROJ_OUT_SPECS,
        out_shape=_PROJ_OUT_SHAPE,
    )(acc, xs, expand, a2d, wq_t, wkv_t, ws_t, bq, bkv, bs)


def _fin2_body(acc_ref, xs_ref, expand_ref, a_ref, x_ref, out_ref):
    acc0 = acc_ref[0]
    acc1 = acc_ref[1]
    msg = jnp.concatenate([acc0[:, :HW], acc1[:, :HW]], axis=1)
    den = jnp.concatenate([acc0[:, HW:HW + HC], acc1[:, HW:HW + HC]], axis=1)
    den_b = jnp.dot(den, expand_ref[...], preferred_element_type=jnp.float32)
    o = msg / (den_b + 1e-16) + xs_ref[...] + x_ref[...]
    out_ref[...] = jnp.where(o >= 0.0, o, o * a_ref[...])


def _fin2_tc(acc, xs, expand, a2d, x):
    return pl.pallas_call(
        _fin2_body,
        grid=(N // BN,),
        in_specs=[
            pl.BlockSpec((NC, BN, D), lambda i: (0, i, 0)),
            pl.BlockSpec((BN, D), lambda i: (i, 0)),
            pl.BlockSpec((H, D), lambda i: (0, 0)),
            pl.BlockSpec((1, D), lambda i: (0, 0)),
            pl.BlockSpec((BN, D), lambda i: (i, 0)),
        ],
        out_specs=pl.BlockSpec((BN, D), lambda i: (i, 0)),
        out_shape=jax.ShapeDtypeStruct((N, D), jnp.float32),
    )(acc, xs, expand, a2d, x)


def kernel(x, edge_index, Wq1, bq1, Wk1, bk1, Wv1, bv1, Ws1, bs1,
           Wq2, bq2, Wk2, bk2, Wv2, bv2, Ws2, bs2, a):
    inv_sqrt_c = 1.0 / (C ** 0.5)
    src = edge_index[0]
    dst = edge_index[1]

    wq1_t = (Wq1.T * inv_sqrt_c).astype(jnp.float32)
    wkv1_t = jnp.concatenate([Wk1.T, Wv1.T], axis=1)
    ws1_t = Ws1.T
    bq1s = (bq1 * inv_sqrt_c).reshape(1, D)
    bkv1 = jnp.concatenate([bk1, bv1]).reshape(1, 2 * D)
    bs1r = bs1.reshape(1, D)

    wq2_t = (Wq2.T * inv_sqrt_c).astype(jnp.float32)
    wkv2_t = jnp.concatenate([Wk2.T, Wv2.T], axis=1)
    ws2_t = Ws2.T
    bq2s = (bq2 * inv_sqrt_c).reshape(1, D)
    bkv2 = jnp.concatenate([bk2, bv2]).reshape(1, 2 * D)
    bs2r = bs2.reshape(1, D)

    expand = jnp.kron(jnp.eye(H, dtype=jnp.float32),
                      jnp.ones((1, C), dtype=jnp.float32))
    a2d = jnp.broadcast_to(a.astype(jnp.float32), (1, D))

    qa1, qb1, kva1, kvb1, xs1 = _proj_tc(x, wq1_t, wkv1_t, ws1_t, bq1s, bkv1, bs1r)
    q1 = jnp.stack([qa1, qb1])
    kv1 = jnp.stack([kva1, kvb1])
    acc1 = _edge_call(q1, kv1, src, dst)
    qa2, qb2, kva2, kvb2, xs2 = _fin1_tc(acc1, xs1, expand, a2d,
                                         wq2_t, wkv2_t, ws2_t, bq2s, bkv2, bs2r)
    q2 = jnp.stack([qa2, qb2])
    kv2 = jnp.stack([kva2, kvb2])
    acc2 = _edge_call(q2, kv2, src, dst)
    return _fin2_tc(acc2, xs2, expand, a2d, x)
